# Initial kernel scaffold; baseline (speedup 1.0000x reference)
import jax
import jax.numpy as jnp
from jax.experimental import pallas as pl


def kernel(atomic_numbers, element_table):
    raise NotImplementedError("write your pallas kernel here")


# speedup vs baseline: 2.4265x; 2.4265x over previous
import functools

import jax
import jax.numpy as jnp
from jax import lax
from jax.experimental import pallas as pl
from jax.experimental.pallas import tpu as pltpu
from jax.experimental.pallas import tpu_sc as plsc

NUM_ATOMS = 100000
NUM_PROPS = 4
TABLE_PAD = 512

NC = 2
NS = 16
NW = NC * NS
LANES = 16

CHUNK = 3136
LAST = NUM_ATOMS - (NW - 1) * CHUNK


def _tile_body(count, base, idx_hbm, out_hbm, table_v, idx_v, rows_v):
    pltpu.sync_copy(idx_hbm.at[pl.ds(base, count)], idx_v.at[pl.ds(0, count)])

    lane = lax.iota(jnp.int32, LANES)
    pos_base = lane * NUM_PROPS

    def group(g, carry):
        z = idx_v[pl.ds(g * LANES, LANES)]
        zb = z * NUM_PROPS
        out_pos = g * (LANES * NUM_PROPS) + pos_base
        for p in range(NUM_PROPS):
            vals = plsc.load_gather(table_v, [zb + p])
            plsc.store_scatter(rows_v, [out_pos + p], vals)
        return carry

    lax.fori_loop(0, count // LANES, group, 0, unroll=4)

    pltpu.sync_copy(
        rows_v.at[pl.ds(0, count * NUM_PROPS)],
        out_hbm.at[pl.ds(base * NUM_PROPS, count * NUM_PROPS)],
    )


@functools.partial(
    pl.kernel,
    mesh=plsc.VectorSubcoreMesh(core_axis_name="c", subcore_axis_name="s"),
    out_type=jax.ShapeDtypeStruct((NUM_ATOMS * NUM_PROPS,), jnp.float32),
    scratch_types=[
        pltpu.VMEM((TABLE_PAD,), jnp.float32),
        pltpu.VMEM((CHUNK,), jnp.int32),
        pltpu.VMEM((CHUNK * NUM_PROPS,), jnp.float32),
    ],
    compiler_params=pltpu.CompilerParams(needs_layout_passes=False),
)
def _sc_lookup(idx_hbm, table_hbm, out_hbm, table_v, idx_v, rows_v):
    wid = lax.axis_index("s") * NC + lax.axis_index("c")
    base = wid * CHUNK

    pltpu.sync_copy(table_hbm, table_v)

    @pl.when(wid < NW - 1)
    def _():
        _tile_body(CHUNK, base, idx_hbm, out_hbm, table_v, idx_v, rows_v)

    @pl.when(wid == NW - 1)
    def _():
        _tile_body(LAST, base, idx_hbm, out_hbm, table_v, idx_v, rows_v)


def kernel(atomic_numbers, element_table):
    idx = atomic_numbers.astype(jnp.int32)
    table_flat = jnp.pad(
        element_table.reshape(-1), (0, TABLE_PAD - element_table.size)
    )
    out = _sc_lookup(idx, table_flat)
    return out.reshape(NUM_ATOMS, NUM_PROPS)


# speedup vs baseline: 9.9957x; 4.1194x over previous
import functools

import jax
import jax.numpy as jnp
from jax import lax
from jax.experimental import pallas as pl
from jax.experimental.pallas import tpu as pltpu
from jax.experimental.pallas import tpu_sc as plsc

NUM_ATOMS = 100000
NUM_ELEMENTS = 119
NUM_PROPS = 4

NC = 2
NS = 16
NW = NC * NS
LANES = 16

BLK = 128
NBLKS = (NUM_ATOMS + BLK - 1) // BLK
ATOMS_PAD = NBLKS * BLK
BLK_WORDS = BLK * NUM_PROPS

BPT = 25
CHUNK = BPT * BLK
LAST = NUM_ATOMS - (NW - 1) * CHUNK
LAST_BLKS = NBLKS - (NW - 1) * BPT


def _tile_body(count, wid, idx_hbm, out_hbm, table_v, idx_v, rows_v):
    base = wid * CHUNK
    pltpu.sync_copy(idx_hbm.at[pl.ds(base, count)], idx_v.at[pl.ds(0, count)])

    lane = lax.iota(jnp.int32, LANES)

    def group(g, carry):
        z = idx_v[pl.ds(g * LANES, LANES)]
        pos0 = (g >> 3) * BLK_WORDS + (g & 7) * LANES + lane
        for p in range(NUM_PROPS):
            vals = plsc.load_gather(table_v, [z + p * BLK])
            plsc.store_scatter(rows_v, [pos0 + p * BLK], vals)
        return carry

    lax.fori_loop(0, count // LANES, group, 0, unroll=4)

    nwords = ((count * NUM_PROPS + BLK_WORDS - 1) // BLK_WORDS) * BLK_WORDS
    pltpu.sync_copy(
        rows_v.at[pl.ds(0, nwords)],
        out_hbm.at[pl.ds(wid * BPT * BLK_WORDS, nwords)],
    )


@functools.partial(
    pl.kernel,
    mesh=plsc.VectorSubcoreMesh(core_axis_name="c", subcore_axis_name="s"),
    out_type=jax.ShapeDtypeStruct((NBLKS * BLK_WORDS,), jnp.float32),
    scratch_types=[
        pltpu.VMEM((NUM_PROPS * BLK,), jnp.float32),
        pltpu.VMEM((CHUNK,), jnp.int32),
        pltpu.VMEM((BPT * BLK_WORDS,), jnp.float32),
    ],
    compiler_params=pltpu.CompilerParams(needs_layout_passes=False),
)
def _sc_lookup(idx_hbm, table_hbm, out_hbm, table_v, idx_v, rows_v):
    wid = lax.axis_index("s") * NC + lax.axis_index("c")

    pltpu.sync_copy(table_hbm, table_v)

    @pl.when(wid < NW - 1)
    def _():
        _tile_body(CHUNK, wid, idx_hbm, out_hbm, table_v, idx_v, rows_v)

    @pl.when(wid == NW - 1)
    def _():
        _tile_body(LAST, wid, idx_hbm, out_hbm, table_v, idx_v, rows_v)


def kernel(atomic_numbers, element_table):
    idx = atomic_numbers.astype(jnp.int32)
    table_pm = jnp.pad(
        element_table.T, ((0, 0), (0, BLK - NUM_ELEMENTS))
    ).reshape(-1)
    out = _sc_lookup(idx, table_pm)
    out = out.reshape(NBLKS, NUM_PROPS, BLK)
    out = jnp.swapaxes(out, 1, 2).reshape(ATOMS_PAD, NUM_PROPS)
    return out[:NUM_ATOMS]


# speedup vs baseline: 11.5070x; 1.1512x over previous
import functools

import jax
import jax.numpy as jnp
from jax import lax
from jax.experimental import pallas as pl
from jax.experimental.pallas import tpu as pltpu
from jax.experimental.pallas import tpu_sc as plsc

NUM_ATOMS = 100000
NUM_ELEMENTS = 119
NUM_PROPS = 4

NC = 2
NS = 16
NW = NC * NS
LANES = 16

BLK = 128
NBLKS = (NUM_ATOMS + BLK - 1) // BLK
ATOMS_PAD = NBLKS * BLK
BLK_WORDS = BLK * NUM_PROPS

BPT = 25
CHUNK = BPT * BLK
LAST = NUM_ATOMS - (NW - 1) * CHUNK
LAST_BLKS = NBLKS - (NW - 1) * BPT


def _tile_body(count, wid, idx_hbm, out_hbm, table_v, idx_v, rows_v):
    base = wid * CHUNK
    pltpu.sync_copy(idx_hbm.at[pl.ds(base, count)], idx_v.at[pl.ds(0, count)])

    lane = lax.iota(jnp.int32, LANES)

    @plsc.parallel_loop(0, count // LANES, unroll=8)
    def group(g):
        z = idx_v[pl.ds(g * LANES, LANES)]
        pos0 = (g >> 3) * BLK_WORDS + (g & 7) * LANES + lane
        for p in range(NUM_PROPS):
            vals = plsc.load_gather(table_v, [z + p * BLK])
            plsc.store_scatter(rows_v, [pos0 + p * BLK], vals)

    nwords = ((count * NUM_PROPS + BLK_WORDS - 1) // BLK_WORDS) * BLK_WORDS
    pltpu.sync_copy(
        rows_v.at[pl.ds(0, nwords)],
        out_hbm.at[pl.ds(wid * BPT * BLK_WORDS, nwords)],
    )


@functools.partial(
    pl.kernel,
    mesh=plsc.VectorSubcoreMesh(core_axis_name="c", subcore_axis_name="s"),
    out_type=jax.ShapeDtypeStruct((NBLKS * BLK_WORDS,), jnp.float32),
    scratch_types=[
        pltpu.VMEM((NUM_PROPS * BLK,), jnp.float32),
        pltpu.VMEM((CHUNK,), jnp.int32),
        pltpu.VMEM((BPT * BLK_WORDS,), jnp.float32),
    ],
    compiler_params=pltpu.CompilerParams(needs_layout_passes=False),
)
def _sc_lookup(idx_hbm, table_hbm, out_hbm, table_v, idx_v, rows_v):
    wid = lax.axis_index("s") * NC + lax.axis_index("c")

    pltpu.sync_copy(table_hbm, table_v)

    @pl.when(wid < NW - 1)
    def _():
        _tile_body(CHUNK, wid, idx_hbm, out_hbm, table_v, idx_v, rows_v)

    @pl.when(wid == NW - 1)
    def _():
        _tile_body(LAST, wid, idx_hbm, out_hbm, table_v, idx_v, rows_v)


def kernel(atomic_numbers, element_table):
    idx = atomic_numbers.astype(jnp.int32)
    table_pm = jnp.pad(
        element_table.T, ((0, 0), (0, BLK - NUM_ELEMENTS))
    ).reshape(-1)
    out = _sc_lookup(idx, table_pm)
    out = out.reshape(NBLKS, NUM_PROPS, BLK)
    out = jnp.swapaxes(out, 1, 2).reshape(ATOMS_PAD, NUM_PROPS)
    return out[:NUM_ATOMS]


# speedup vs baseline: 11.5785x; 1.0062x over previous
import functools

import jax
import jax.numpy as jnp
from jax import lax
from jax.experimental import pallas as pl
from jax.experimental.pallas import tpu as pltpu
from jax.experimental.pallas import tpu_sc as plsc

NUM_ATOMS = 100000
NUM_ELEMENTS = 119
NUM_PROPS = 4

NC = 2
NS = 16
NW = NC * NS
LANES = 16

BLK = 128
NBLKS = (NUM_ATOMS + BLK - 1) // BLK
ATOMS_PAD = NBLKS * BLK
BLK_WORDS = BLK * NUM_PROPS

BPT = 25
CHUNK = BPT * BLK
LAST = NUM_ATOMS - (NW - 1) * CHUNK
LAST_BLKS = NBLKS - (NW - 1) * BPT


def _gather_groups(lo, hi, idx_v, table_v, rows_v):
    @plsc.parallel_loop(lo, hi, unroll=8)
    def group(g):
        z = idx_v[pl.ds(g * LANES, LANES)]
        off = (g >> 3) * BLK_WORDS + (g & 7) * LANES
        for p in range(NUM_PROPS):
            vals = plsc.load_gather(table_v.at[pl.ds(p * BLK, BLK)], [z])
            rows_v[pl.ds(off + p * BLK, LANES)] = vals


def _tile_body(count, wid, idx_hbm, out_hbm, table_v, idx_v, rows_v,
               sem_i, sem_a, sem_b):
    base = wid * CHUNK
    pltpu.async_copy(
        idx_hbm.at[pl.ds(base, count)], idx_v.at[pl.ds(0, count)], sem_i
    ).wait()

    ngroups = count // LANES
    out_base = wid * BPT * BLK_WORDS
    if count == CHUNK:
        half_g = (ngroups // 2) // 8 * 8
        half_w = (half_g // 8) * BLK_WORDS
        nwords = (ngroups // 8) * BLK_WORDS
        _gather_groups(0, half_g, idx_v, table_v, rows_v)
        d1 = pltpu.async_copy(
            rows_v.at[pl.ds(0, half_w)],
            out_hbm.at[pl.ds(out_base, half_w)],
            sem_a,
        )
        _gather_groups(half_g, ngroups, idx_v, table_v, rows_v)
        d2 = pltpu.async_copy(
            rows_v.at[pl.ds(half_w, nwords - half_w)],
            out_hbm.at[pl.ds(out_base + half_w, nwords - half_w)],
            sem_b,
        )
        d1.wait()
        d2.wait()
    else:
        nwords = ((count * NUM_PROPS + BLK_WORDS - 1) // BLK_WORDS) * BLK_WORDS
        _gather_groups(0, ngroups, idx_v, table_v, rows_v)
        pltpu.async_copy(
            rows_v.at[pl.ds(0, nwords)],
            out_hbm.at[pl.ds(out_base, nwords)],
            sem_a,
        ).wait()


@functools.partial(
    pl.kernel,
    mesh=plsc.VectorSubcoreMesh(core_axis_name="c", subcore_axis_name="s"),
    out_type=jax.ShapeDtypeStruct((NBLKS * BLK_WORDS,), jnp.float32),
    scratch_types=[
        pltpu.VMEM((NUM_PROPS * BLK,), jnp.float32),
        pltpu.VMEM((CHUNK,), jnp.int32),
        pltpu.VMEM((BPT * BLK_WORDS,), jnp.float32),
        pltpu.SemaphoreType.DMA,
        pltpu.SemaphoreType.DMA,
        pltpu.SemaphoreType.DMA,
        pltpu.SemaphoreType.DMA,
    ],
    compiler_params=pltpu.CompilerParams(needs_layout_passes=False),
)
def _sc_lookup(idx_hbm, table_hbm, out_hbm, table_v, idx_v, rows_v,
               sem_t, sem_i, sem_a, sem_b):
    wid = lax.axis_index("s") * NC + lax.axis_index("c")

    pltpu.async_copy(table_hbm, table_v, sem_t).wait()

    @pl.when(wid < NW - 1)
    def _():
        _tile_body(CHUNK, wid, idx_hbm, out_hbm, table_v, idx_v, rows_v,
                   sem_i, sem_a, sem_b)

    @pl.when(wid == NW - 1)
    def _():
        _tile_body(LAST, wid, idx_hbm, out_hbm, table_v, idx_v, rows_v,
                   sem_i, sem_a, sem_b)


def kernel(atomic_numbers, element_table):
    idx = atomic_numbers.astype(jnp.int32)
    table_pm = jnp.pad(
        element_table.T, ((0, 0), (0, BLK - NUM_ELEMENTS))
    ).reshape(-1)
    out = _sc_lookup(idx, table_pm)
    out = out.reshape(NBLKS, NUM_PROPS, BLK)
    out = jnp.swapaxes(out, 1, 2).reshape(ATOMS_PAD, NUM_PROPS)
    return out[:NUM_ATOMS]


# speedup vs baseline: 11.7275x; 1.0129x over previous
import functools

import jax
import jax.numpy as jnp
from jax import lax
from jax.experimental import pallas as pl
from jax.experimental.pallas import tpu as pltpu
from jax.experimental.pallas import tpu_sc as plsc

NUM_ATOMS = 100000
NUM_ELEMENTS = 119
NUM_PROPS = 4

NC = 2
NS = 16
NW = NC * NS
LANES = 16

BLK = 128
NBLKS = (NUM_ATOMS + BLK - 1) // BLK
ATOMS_PAD = NBLKS * BLK
BLK_WORDS = BLK * NUM_PROPS

BPT = 25
CHUNK = BPT * BLK
LAST = NUM_ATOMS - (NW - 1) * CHUNK
LAST_BLKS = NBLKS - (NW - 1) * BPT


def _gather_groups(lo, hi, idx_v, table_v, rows_v):
    @plsc.parallel_loop(lo, hi, unroll=8)
    def group(g):
        z = idx_v[pl.ds(g * LANES, LANES)]
        off = (g >> 3) * BLK_WORDS + (g & 7) * LANES
        for p in range(NUM_PROPS):
            vals = plsc.load_gather(table_v.at[pl.ds(p * BLK, BLK)], [z])
            rows_v[pl.ds(off + p * BLK, LANES)] = vals


def _tile_body(count, wid, idx_hbm, table_hbm, out_hbm, table_v, idx_v,
               rows_v, sem_t, sem_i, sem_a, sem_b):
    base = wid * CHUNK
    d_i = pltpu.async_copy(
        idx_hbm.at[pl.ds(base, count)], idx_v.at[pl.ds(0, count)], sem_i
    )
    d_t = pltpu.async_copy(table_hbm, table_v, sem_t)
    d_t.wait()
    d_i.wait()

    ngroups = count // LANES
    out_base = wid * BPT * BLK_WORDS
    if count == CHUNK:
        half_g = (ngroups // 2) // 8 * 8
        half_w = (half_g // 8) * BLK_WORDS
        nwords = (ngroups // 8) * BLK_WORDS
        _gather_groups(0, half_g, idx_v, table_v, rows_v)
        d1 = pltpu.async_copy(
            rows_v.at[pl.ds(0, half_w)],
            out_hbm.at[pl.ds(out_base, half_w)],
            sem_a,
        )
        _gather_groups(half_g, ngroups, idx_v, table_v, rows_v)
        d2 = pltpu.async_copy(
            rows_v.at[pl.ds(half_w, nwords - half_w)],
            out_hbm.at[pl.ds(out_base + half_w, nwords - half_w)],
            sem_b,
        )
        d1.wait()
        d2.wait()
    else:
        nwords = ((count * NUM_PROPS + BLK_WORDS - 1) // BLK_WORDS) * BLK_WORDS
        _gather_groups(0, ngroups, idx_v, table_v, rows_v)
        pltpu.async_copy(
            rows_v.at[pl.ds(0, nwords)],
            out_hbm.at[pl.ds(out_base, nwords)],
            sem_a,
        ).wait()


@functools.partial(
    pl.kernel,
    mesh=plsc.VectorSubcoreMesh(core_axis_name="c", subcore_axis_name="s"),
    out_type=jax.ShapeDtypeStruct((NBLKS * BLK_WORDS,), jnp.float32),
    scratch_types=[
        pltpu.VMEM((NUM_PROPS * BLK,), jnp.float32),
        pltpu.VMEM((CHUNK,), jnp.int32),
        pltpu.VMEM((BPT * BLK_WORDS,), jnp.float32),
        pltpu.SemaphoreType.DMA,
        pltpu.SemaphoreType.DMA,
        pltpu.SemaphoreType.DMA,
        pltpu.SemaphoreType.DMA,
    ],
    compiler_params=pltpu.CompilerParams(needs_layout_passes=False),
)
def _sc_lookup(idx_hbm, table_hbm, out_hbm, table_v, idx_v, rows_v,
               sem_t, sem_i, sem_a, sem_b):
    wid = lax.axis_index("s") * NC + lax.axis_index("c")

    @pl.when(wid < NW - 1)
    def _():
        _tile_body(CHUNK, wid, idx_hbm, table_hbm, out_hbm, table_v, idx_v,
                   rows_v, sem_t, sem_i, sem_a, sem_b)

    @pl.when(wid == NW - 1)
    def _():
        _tile_body(LAST, wid, idx_hbm, table_hbm, out_hbm, table_v, idx_v,
                   rows_v, sem_t, sem_i, sem_a, sem_b)


def kernel(atomic_numbers, element_table):
    idx = atomic_numbers.astype(jnp.int32)
    table_pm = jnp.pad(
        element_table.T, ((0, 0), (0, BLK - NUM_ELEMENTS))
    ).reshape(-1)
    out = _sc_lookup(idx, table_pm)
    out = out.reshape(NBLKS, NUM_PROPS, BLK)
    out = jnp.swapaxes(out, 1, 2).reshape(ATOMS_PAD, NUM_PROPS)
    return out[:NUM_ATOMS]
